# baseline (speedup 1.0000x reference)
import jax
import jax.numpy as jnp
from jax.experimental import pallas as pl
from jax.experimental.pallas import tpu as pltpu

N = 7
NP = 8
E = 32
F0 = 224
H1 = 64
H2 = 256
OUT = 576


def _gnn_kernel(ei_ref, xT_ref, w1_ref, b1_ref, w2_ref, b2_ref,
                wfc_ref, bfc_ref, out_ref, h2_ref):
    f32 = jnp.float32
    i = pl.program_id(0)

    @pl.when(i == 0)
    def _prologue():
        row = ei_ref[0:1, :E]
        col = ei_ref[1:2, :E]
        nodes = jax.lax.broadcasted_iota(jnp.int32, (NP, 1), 0)
        ohr = (nodes == row).astype(f32)
        ohc = (nodes == col).astype(f32)
        real = (nodes < N).astype(f32)
        deg = jnp.sum(ohc, axis=1, keepdims=True) + real
        dinv = jnp.where(deg > 0, jax.lax.rsqrt(jnp.maximum(deg, 1e-12)), 0.0)
        dinv_row = jnp.sum(ohr * dinv, axis=0, keepdims=True)
        dinv_col = jnp.sum(ohc * dinv, axis=0, keepdims=True)
        norm = dinv_row * dinv_col
        A = jax.lax.dot_general(ohc * norm, ohr,
                                (((1,), (1,)), ((), ())),
                                preferred_element_type=f32)
        eye = (nodes == jax.lax.broadcasted_iota(jnp.int32, (1, NP), 1)
               ).astype(f32)
        A = A + eye * (dinv * dinv) * real

        xw1 = jnp.dot(xT_ref[...], w1_ref[...], preferred_element_type=f32)
        h1 = jax.nn.relu(jnp.dot(A, xw1, preferred_element_type=f32)
                         + b1_ref[...])
        xw2 = jnp.dot(h1, w2_ref[...], preferred_element_type=f32)
        h2_ref[...] = jax.nn.relu(jnp.dot(A, xw2, preferred_element_type=f32)
                                  + b2_ref[...])
        out_ref[...] = bfc_ref[...]

    out_ref[...] += jnp.dot(h2_ref[pl.ds(i, 1), :], wfc_ref[0],
                            preferred_element_type=f32)


def kernel(x, edge_index, W1, b1, W2, b2, Wfc, bfc):
    xT = jnp.pad(x.T, ((0, NP - N), (0, 0)))
    ei = jnp.pad(edge_index.astype(jnp.int32), ((0, 6), (0, 96)))
    out = pl.pallas_call(
        _gnn_kernel,
        grid=(N,),
        in_specs=[
            pl.BlockSpec((8, 128), lambda i: (0, 0)),
            pl.BlockSpec((NP, F0), lambda i: (0, 0)),
            pl.BlockSpec((F0, H1), lambda i: (0, 0)),
            pl.BlockSpec((1, H1), lambda i: (0, 0)),
            pl.BlockSpec((H1, H2), lambda i: (0, 0)),
            pl.BlockSpec((1, H2), lambda i: (0, 0)),
            pl.BlockSpec((1, H2, OUT), lambda i: (i, 0, 0)),
            pl.BlockSpec((1, OUT), lambda i: (0, 0)),
        ],
        out_specs=pl.BlockSpec((1, OUT), lambda i: (0, 0)),
        out_shape=jax.ShapeDtypeStruct((1, OUT), jnp.float32),
        scratch_shapes=[pltpu.VMEM((NP, H2), jnp.float32)],
    )(ei, xT, W1, b1.reshape(1, H1), W2, b2.reshape(1, H2),
      Wfc.reshape(N, H2, OUT), bfc.reshape(1, OUT))
    return out.reshape(24, 24)


# speedup vs baseline: 1.2247x; 1.2247x over previous
import jax
import jax.numpy as jnp
from jax.experimental import pallas as pl
from jax.experimental.pallas import tpu as pltpu

N = 7
NP = 8
E = 32
F0 = 224
H1 = 64
H2 = 256
OUT = 576


def _gnn_kernel(ei_hbm, x_hbm, w1_hbm, b1_hbm, w2_hbm, b2_hbm,
                wfc_hbm, bfc_hbm, out_ref,
                ei_s, x_s, w1_s, b1_s, w2_s, b2_s, wfc_s, bfc_s, sems):
    f32 = jnp.float32
    cps = [
        pltpu.make_async_copy(wfc_hbm, wfc_s, sems.at[0]),
        pltpu.make_async_copy(ei_hbm, ei_s.at[pl.ds(0, 2), pl.ds(0, E)],
                              sems.at[1]),
        pltpu.make_async_copy(x_hbm, x_s, sems.at[2]),
        pltpu.make_async_copy(w1_hbm, w1_s, sems.at[3]),
        pltpu.make_async_copy(b1_hbm, b1_s, sems.at[4]),
        pltpu.make_async_copy(w2_hbm, w2_s, sems.at[5]),
        pltpu.make_async_copy(b2_hbm, b2_s, sems.at[6]),
        pltpu.make_async_copy(bfc_hbm, bfc_s, sems.at[7]),
    ]
    for c in cps:
        c.start()
    for c in cps[1:]:
        c.wait()

    row = ei_s[0:1, :E]
    col = ei_s[1:2, :E]
    nodes = jax.lax.broadcasted_iota(jnp.int32, (NP, 1), 0)
    ohr = (nodes == row).astype(f32)
    ohc = (nodes == col).astype(f32)
    real = (nodes < N).astype(f32)
    deg = jnp.sum(ohc, axis=1, keepdims=True) + real
    dinv = jnp.where(deg > 0, jax.lax.rsqrt(jnp.maximum(deg, 1e-12)), 0.0)
    dinv_row = jnp.sum(ohr * dinv, axis=0, keepdims=True)
    dinv_col = jnp.sum(ohc * dinv, axis=0, keepdims=True)
    norm = dinv_row * dinv_col
    A = jax.lax.dot_general(ohc * norm, ohr,
                            (((1,), (1,)), ((), ())),
                            preferred_element_type=f32)
    eye = (nodes == jax.lax.broadcasted_iota(jnp.int32, (1, NP), 1)
           ).astype(f32)
    A = A + eye * (dinv * dinv) * real

    xw1 = jax.lax.dot_general(x_s[...], w1_s[...],
                              (((0,), (0,)), ((), ())),
                              preferred_element_type=f32)
    xw1 = jnp.pad(xw1, ((0, NP - N), (0, 0)))
    h1 = jax.nn.relu(jnp.dot(A, xw1, preferred_element_type=f32)
                     + b1_s[...])
    xw2 = jnp.dot(h1, w2_s[...], preferred_element_type=f32)
    h2 = jax.nn.relu(jnp.dot(A, xw2, preferred_element_type=f32)
                     + b2_s[...])

    cps[0].wait()
    acc = bfc_s[...]
    for n in range(N):
        acc = acc + jnp.dot(h2[n:n + 1, :],
                            wfc_s[pl.ds(n * H2, H2), :],
                            preferred_element_type=f32)
    out_ref[...] = acc


def kernel(x, edge_index, W1, b1, W2, b2, Wfc, bfc):
    any_spec = pl.BlockSpec(memory_space=pl.ANY)
    out = pl.pallas_call(
        _gnn_kernel,
        in_specs=[any_spec] * 8,
        out_specs=pl.BlockSpec((1, OUT), lambda: (0, 0)),
        out_shape=jax.ShapeDtypeStruct((1, OUT), jnp.float32),
        scratch_shapes=[
            pltpu.VMEM((8, E), jnp.int32),
            pltpu.VMEM((F0, N), jnp.float32),
            pltpu.VMEM((F0, H1), jnp.float32),
            pltpu.VMEM((1, H1), jnp.float32),
            pltpu.VMEM((H1, H2), jnp.float32),
            pltpu.VMEM((1, H2), jnp.float32),
            pltpu.VMEM((N * H2, OUT), jnp.float32),
            pltpu.VMEM((1, OUT), jnp.float32),
            pltpu.SemaphoreType.DMA((8,)),
        ],
    )(edge_index, x, W1, b1.reshape(1, H1), W2, b2.reshape(1, H2),
      Wfc, bfc.reshape(1, OUT))
    return out.reshape(24, 24)
